# baseline (speedup 1.0000x reference)
import functools

import jax
import jax.numpy as jnp
from jax import lax
from jax.experimental import pallas as pl
from jax.experimental.pallas import tpu as pltpu
from jax.experimental.pallas import tpu_sc as plsc

N = 50000
T = 50
C = 1000
E = 3200000
NP = 50176
NTILES = 32
EPT = E // NTILES
BLK = 4000
NB = EPT // BLK
GRP = BLK // 16
EROWS = 3125


def _retina_body(x_ref, w_ref, g_ref, o_ref):
    a = w_ref[0] + g_ref[0]
    m = jnp.max(a, axis=0, keepdims=True)
    p = jnp.exp(a - m)
    s = jnp.sum(p, axis=0, keepdims=True)
    v = x_ref[0] / s
    o_ref[0] = lax.dot_general(v, p, (((1,), (1,)), ((), ())),
                               preferred_element_type=jnp.float32)


def _retina(x3, w, g):
    return pl.pallas_call(
        _retina_body,
        grid=(T,),
        in_specs=[pl.BlockSpec((1, 1, C), lambda t: (t, 0, 0)),
                  pl.BlockSpec((1, C, C), lambda t: (t, 0, 0)),
                  pl.BlockSpec((1, C, C), lambda t: (t, 0, 0))],
        out_specs=pl.BlockSpec((1, 1, C), lambda t: (t, 0, 0)),
        out_shape=jax.ShapeDtypeStruct((T, 1, C), jnp.float32),
    )(x3, w, g)


def _nz_body(h_ref, a_ref, o_ref, w_ref):
    h = h_ref[...]
    mask = h != 0.0
    cnt = jnp.sum(mask.astype(jnp.float32))
    mean = jnp.sum(jnp.where(mask, h, 0.0)) / cnt
    var = jnp.sum(jnp.where(mask, (h - mean) ** 2, 0.0)) / (cnt - 1.0)
    o_ref[...] = jnp.where(mask, (h - mean) / jnp.sqrt(var), h)
    w_ref[...] = jnp.log1p(a_ref[...])


def _normalize_nz_log1p(h, a2):
    return pl.pallas_call(
        _nz_body,
        out_shape=(jax.ShapeDtypeStruct((T, C), jnp.float32),
                   jax.ShapeDtypeStruct((EROWS, 1024), jnp.float32)),
    )(h, a2)


def _sc_pass(h_pad, src, dst, ew):
    mesh = plsc.VectorSubcoreMesh(core_axis_name="c", subcore_axis_name="s",
                                  num_cores=2, num_subcores=16)

    @functools.partial(
        pl.kernel,
        out_type=jax.ShapeDtypeStruct((NTILES, NP), jnp.float32),
        mesh=mesh,
        compiler_params=pltpu.CompilerParams(needs_layout_passes=False),
        scratch_types=[
            pltpu.VMEM((NP,), jnp.float32),
            pltpu.VMEM((NP,), jnp.float32),
            pltpu.VMEM((BLK,), jnp.int32),
            pltpu.VMEM((BLK,), jnp.int32),
            pltpu.VMEM((BLK,), jnp.float32),
            pltpu.VMEM((BLK,), jnp.int32),
            pltpu.VMEM((BLK,), jnp.int32),
            pltpu.VMEM((BLK,), jnp.float32),
            pltpu.SemaphoreType.DMA,
            pltpu.SemaphoreType.DMA,
        ],
    )
    def sc_kernel(h_hbm, src_hbm, dst_hbm, w_hbm, out_hbm,
                  h_l, acc, sb0, db0, wb0, sb1, db1, wb1, sem0, sem1):
        cid = lax.axis_index("c")
        sid = lax.axis_index("s")
        wid = sid * 2 + cid

        pltpu.sync_copy(h_hbm, h_l.at[pl.ds(0, N)])

        zeros = jnp.zeros((16,), jnp.float32)

        @plsc.parallel_loop(0, NP, step=16, unroll=16)
        def _zero(i):
            acc[pl.ds(i, 16)] = zeros

        bufs = ((sb0, db0, wb0, sem0), (sb1, db1, wb1, sem1))

        def start(b, bs):
            sbb, dbb, wbb, sem = bs
            base = pl.multiple_of(wid * EPT + b * BLK, 8)
            return (pltpu.async_copy(src_hbm.at[pl.ds(base, BLK)], sbb, sem),
                    pltpu.async_copy(dst_hbm.at[pl.ds(base, BLK)], dbb, sem),
                    pltpu.async_copy(w_hbm.at[pl.ds(base, BLK)], wbb, sem))

        cps = start(0, bufs[0])
        for b in range(NB):
            nxt = start(b + 1, bufs[(b + 1) % 2]) if b + 1 < NB else None
            for cp in cps:
                cp.wait()
            sbb, dbb, wbb, _ = bufs[b % 2]

            @plsc.parallel_loop(0, BLK, step=16, unroll=10)
            def _edges(e):
                si = sbb[pl.ds(e, 16)]
                di = dbb[pl.ds(e, 16)]
                wv = wbb[pl.ds(e, 16)]
                vals = plsc.load_gather(h_l, [si])
                plsc.addupdate_scatter(acc, [di], vals * wv)
            cps = nxt

        pltpu.sync_copy(acc, out_hbm.at[wid])

    return sc_kernel(h_pad, src, dst, ew)


def _ln_body(p_ref, g_ref, b_ref, o_ref):
    p = p_ref[...]
    h = jnp.sum(p, axis=0, keepdims=True)
    mu = jnp.sum(h) / float(N)
    var = jnp.sum(h * h) / float(N) - mu * mu
    hn = (h - mu) * lax.rsqrt(var + 1e-5)
    o_ref[...] = hn[:, :N] * g_ref[...] + b_ref[...]


def _layernorm(p, g2, b2):
    return pl.pallas_call(
        _ln_body,
        out_shape=jax.ShapeDtypeStruct((1, N), jnp.float32),
    )(p, g2, b2)


def _head_body(hs_ref, d_ref, fw_ref, fb_ref, o_ref):
    v = hs_ref[...][:, 0:1]
    dv = d_ref[...][:, 0:1]
    sel = jnp.where(dv == 1.0, v, 0.0)
    mn = jnp.min(sel)
    mx = jnp.max(sel)
    sel = (sel - mn) / (mx - mn)
    m = jnp.sum(sel) / float(N // 10)
    y = m * fw_ref[0, 0] + fb_ref[0, 0]
    o_ref[...] = jnp.broadcast_to(jnp.maximum(y, 0.0), (1, 1))


def _head(hs, d2, fw, fb):
    return pl.pallas_call(
        _head_body,
        out_shape=jax.ShapeDtypeStruct((1, 1), jnp.float32),
    )(hs, d2, fw, fb)


def kernel(x, edge_index, edge_attr, W_retina, gumbel, ln_gamma, ln_beta,
           fc_w, fc_b, decision):
    x3 = x.reshape(T, 1, C)
    h_tc = _retina(x3, W_retina, gumbel).reshape(T, C)
    h_nz, ew2 = _normalize_nz_log1p(h_tc, edge_attr.reshape(EROWS, 1024))
    ew = ew2.reshape(E)

    src = edge_index[0]
    dst = edge_index[1]
    g2 = ln_gamma.reshape(1, N)
    b2 = ln_beta.reshape(1, N)

    p1 = _sc_pass(h_nz.reshape(N), src, dst, ew)
    h1 = _layernorm(p1, g2, b2)
    p2 = _sc_pass(h1.reshape(N), src, dst, ew)
    h2 = _layernorm(p2, g2, b2)

    hs = h2.reshape(N // 10, 10)
    d2 = decision.reshape(N // 10, 10)
    y = _head(hs, d2, fc_w, fc_b.reshape(1, 1))
    return y.reshape(1)


# speedup vs baseline: 1.1304x; 1.1304x over previous
import functools

import jax
import jax.numpy as jnp
from jax import lax
from jax.experimental import pallas as pl
from jax.experimental.pallas import tpu as pltpu
from jax.experimental.pallas import tpu_sc as plsc

N = 50000
T = 50
C = 1000
E = 3200000
NP = 50176
NTILES = 32
EPT = E // NTILES
BLK = 4000
NB = EPT // BLK
GRP = BLK // 16
EROWS = 3125


def _retina_body(x_ref, w_ref, g_ref, ea_ref, o_ref, ew_ref, h_s):
    t = pl.program_id(0)
    a = w_ref[0] + g_ref[0]
    m = jnp.max(a, axis=0, keepdims=True)
    p = jnp.exp(a - m)
    s = jnp.sum(p, axis=0, keepdims=True)
    v = x_ref[0] / s
    h_s[pl.ds(t, 1), :] = lax.dot_general(v, p, (((1,), (1,)), ((), ())),
                                          preferred_element_type=jnp.float32)
    ew_ref[0] = jnp.log1p(ea_ref[0])

    @pl.when(t == T - 1)
    def _():
        h = h_s[...]
        mask = h != 0.0
        cnt = jnp.sum(mask.astype(jnp.float32))
        mean = jnp.sum(jnp.where(mask, h, 0.0)) / cnt
        var = jnp.sum(jnp.where(mask, (h - mean) ** 2, 0.0)) / (cnt - 1.0)
        o_ref[...] = jnp.where(mask, (h - mean) / jnp.sqrt(var), h)


def _retina(x3, w, g, ea):
    return pl.pallas_call(
        _retina_body,
        grid=(T,),
        in_specs=[pl.BlockSpec((1, 1, C), lambda t: (t, 0, 0)),
                  pl.BlockSpec((1, C, C), lambda t: (t, 0, 0)),
                  pl.BlockSpec((1, C, C), lambda t: (t, 0, 0)),
                  pl.BlockSpec((1, 1, E // T), lambda t: (t, 0, 0))],
        out_specs=(pl.BlockSpec((T, C), lambda t: (0, 0)),
                   pl.BlockSpec((1, 1, E // T), lambda t: (t, 0, 0))),
        out_shape=(jax.ShapeDtypeStruct((T, C), jnp.float32),
                   jax.ShapeDtypeStruct((T, 1, E // T), jnp.float32)),
        scratch_shapes=[pltpu.VMEM((T, C), jnp.float32)],
    )(x3, w, g, ea)


def _sc_pass(h_pad, src, dst, ew):
    mesh = plsc.VectorSubcoreMesh(core_axis_name="c", subcore_axis_name="s",
                                  num_cores=2, num_subcores=16)

    @functools.partial(
        pl.kernel,
        out_type=jax.ShapeDtypeStruct((NTILES, NP), jnp.float32),
        mesh=mesh,
        compiler_params=pltpu.CompilerParams(needs_layout_passes=False),
        scratch_types=[
            pltpu.VMEM((NP,), jnp.float32),
            pltpu.VMEM((NP,), jnp.float32),
            pltpu.VMEM((BLK,), jnp.int32),
            pltpu.VMEM((BLK,), jnp.int32),
            pltpu.VMEM((BLK,), jnp.float32),
            pltpu.VMEM((BLK,), jnp.int32),
            pltpu.VMEM((BLK,), jnp.int32),
            pltpu.VMEM((BLK,), jnp.float32),
            pltpu.SemaphoreType.DMA,
            pltpu.SemaphoreType.DMA,
        ],
    )
    def sc_kernel(h_hbm, src_hbm, dst_hbm, w_hbm, out_hbm,
                  h_l, acc, sb0, db0, wb0, sb1, db1, wb1, sem0, sem1):
        cid = lax.axis_index("c")
        sid = lax.axis_index("s")
        wid = sid * 2 + cid

        pltpu.sync_copy(h_hbm, h_l.at[pl.ds(0, N)])

        zeros = jnp.zeros((16,), jnp.float32)

        @plsc.parallel_loop(0, NP, step=16, unroll=16)
        def _zero(i):
            acc[pl.ds(i, 16)] = zeros

        bufs = ((sb0, db0, wb0, sem0), (sb1, db1, wb1, sem1))

        def start(b, bs):
            sbb, dbb, wbb, sem = bs
            base = pl.multiple_of(wid * EPT + b * BLK, 8)
            return (pltpu.async_copy(src_hbm.at[pl.ds(base, BLK)], sbb, sem),
                    pltpu.async_copy(dst_hbm.at[pl.ds(base, BLK)], dbb, sem),
                    pltpu.async_copy(w_hbm.at[pl.ds(base, BLK)], wbb, sem))

        cps = start(0, bufs[0])
        for b in range(NB):
            nxt = start(b + 1, bufs[(b + 1) % 2]) if b + 1 < NB else None
            for cp in cps:
                cp.wait()
            sbb, dbb, wbb, _ = bufs[b % 2]

            @plsc.parallel_loop(0, BLK, step=16, unroll=10)
            def _edges(e):
                si = sbb[pl.ds(e, 16)]
                di = dbb[pl.ds(e, 16)]
                wv = wbb[pl.ds(e, 16)]
                vals = plsc.load_gather(h_l, [si])
                plsc.addupdate_scatter(acc, [di], vals * wv)
            cps = nxt

        pltpu.sync_copy(acc, out_hbm.at[wid])

    return sc_kernel(h_pad, src, dst, ew)


def _ln_body(p_ref, g_ref, b_ref, o_ref):
    p = p_ref[...]
    h = jnp.sum(p, axis=0, keepdims=True)
    mu = jnp.sum(h) / float(N)
    var = jnp.sum(h * h) / float(N) - mu * mu
    hn = (h - mu) * lax.rsqrt(var + 1e-5)
    o_ref[...] = hn[:, :N] * g_ref[...] + b_ref[...]


def _layernorm(p, g2, b2):
    return pl.pallas_call(
        _ln_body,
        out_shape=jax.ShapeDtypeStruct((1, N), jnp.float32),
    )(p, g2, b2)


def _head_body(hs_ref, d_ref, fw_ref, fb_ref, o_ref):
    v = hs_ref[...][:, 0:1]
    dv = d_ref[...][:, 0:1]
    sel = jnp.where(dv == 1.0, v, 0.0)
    mn = jnp.min(sel)
    mx = jnp.max(sel)
    sel = (sel - mn) / (mx - mn)
    m = jnp.sum(sel) / float(N // 10)
    y = m * fw_ref[0, 0] + fb_ref[0, 0]
    o_ref[...] = jnp.broadcast_to(jnp.maximum(y, 0.0), (1, 1))


def _head(hs, d2, fw, fb):
    return pl.pallas_call(
        _head_body,
        out_shape=jax.ShapeDtypeStruct((1, 1), jnp.float32),
    )(hs, d2, fw, fb)


def kernel(x, edge_index, edge_attr, W_retina, gumbel, ln_gamma, ln_beta,
           fc_w, fc_b, decision):
    x3 = x.reshape(T, 1, C)
    h_nz, ew2 = _retina(x3, W_retina, gumbel, edge_attr.reshape(T, 1, E // T))
    ew = ew2.reshape(E)

    src = edge_index[0]
    dst = edge_index[1]
    g2 = ln_gamma.reshape(1, N)
    b2 = ln_beta.reshape(1, N)

    p1 = _sc_pass(h_nz.reshape(N), src, dst, ew)
    h1 = _layernorm(p1, g2, b2)
    p2 = _sc_pass(h1.reshape(N), src, dst, ew)
    h2 = _layernorm(p2, g2, b2)

    hs = h2.reshape(N // 10, 10)
    d2 = decision.reshape(N // 10, 10)
    y = _head(hs, d2, fc_w, fc_b.reshape(1, 1))
    return y.reshape(1)


# speedup vs baseline: 1.1874x; 1.0505x over previous
import functools

import jax
import jax.numpy as jnp
from jax import lax
from jax.experimental import pallas as pl
from jax.experimental.pallas import tpu as pltpu
from jax.experimental.pallas import tpu_sc as plsc

N = 50000
T = 50
C = 1000
E = 3200000
NP = 50176
NTILES = 32
EPT = E // NTILES
BLK = 4000
NB = EPT // BLK
GRP = BLK // 16
EROWS = 3125


def _retina_body(x_ref, w_ref, g_ref, ea_ref, o_ref, ew_ref, h_s):
    t = pl.program_id(0)
    a = w_ref[0] + g_ref[0]
    m = jnp.max(a, axis=0, keepdims=True)
    p = jnp.exp(a - m)
    s = jnp.sum(p, axis=0, keepdims=True)
    v = x_ref[0] / s
    h_s[pl.ds(t, 1), :] = lax.dot_general(v, p, (((1,), (1,)), ((), ())),
                                          preferred_element_type=jnp.float32)
    lw = jnp.log1p(ea_ref[0].reshape(E // T))

    @pl.when(t % 2 == 0)
    def _():
        ew_ref[pl.ds(0, E // T)] = lw

    @pl.when(t % 2 == 1)
    def _():
        ew_ref[pl.ds(E // T, E // T)] = lw

    @pl.when(t == T - 1)
    def _():
        h = h_s[...]
        mask = h != 0.0
        cnt = jnp.sum(mask.astype(jnp.float32))
        mean = jnp.sum(jnp.where(mask, h, 0.0)) / cnt
        var = jnp.sum(jnp.where(mask, (h - mean) ** 2, 0.0)) / (cnt - 1.0)
        o_ref[...] = jnp.where(mask, (h - mean) / jnp.sqrt(var), h)


def _retina(x3, w, g, ea):
    return pl.pallas_call(
        _retina_body,
        grid=(T,),
        in_specs=[pl.BlockSpec((1, 1, C), lambda t: (t, 0, 0)),
                  pl.BlockSpec((1, C, C), lambda t: (t, 0, 0)),
                  pl.BlockSpec((1, C, C), lambda t: (t, 0, 0)),
                  pl.BlockSpec((1, 1, E // T), lambda t: (t, 0, 0))],
        out_specs=(pl.BlockSpec((T, C), lambda t: (0, 0)),
                   pl.BlockSpec((2 * E // T,), lambda t: (t // 2,))),
        out_shape=(jax.ShapeDtypeStruct((T, C), jnp.float32),
                   jax.ShapeDtypeStruct((E,), jnp.float32)),
        scratch_shapes=[pltpu.VMEM((T, C), jnp.float32)],
    )(x3, w, g, ea)


def _sc_pass(h_pad, edge_index, ew):
    mesh = plsc.VectorSubcoreMesh(core_axis_name="c", subcore_axis_name="s",
                                  num_cores=2, num_subcores=16)

    @functools.partial(
        pl.kernel,
        out_type=jax.ShapeDtypeStruct((NTILES, NP), jnp.float32),
        mesh=mesh,
        compiler_params=pltpu.CompilerParams(needs_layout_passes=False),
        scratch_types=[
            pltpu.VMEM((NP,), jnp.float32),
            pltpu.VMEM((NP,), jnp.float32),
            pltpu.VMEM((BLK,), jnp.int32),
            pltpu.VMEM((BLK,), jnp.int32),
            pltpu.VMEM((BLK,), jnp.float32),
            pltpu.VMEM((BLK,), jnp.int32),
            pltpu.VMEM((BLK,), jnp.int32),
            pltpu.VMEM((BLK,), jnp.float32),
            pltpu.SemaphoreType.DMA,
            pltpu.SemaphoreType.DMA,
        ],
    )
    def sc_kernel(h_hbm, ei_hbm, w_hbm, out_hbm,
                  h_l, acc, sb0, db0, wb0, sb1, db1, wb1, sem0, sem1):
        cid = lax.axis_index("c")
        sid = lax.axis_index("s")
        wid = sid * 2 + cid

        bufs = ((sb0, db0, wb0, sem0), (sb1, db1, wb1, sem1))

        def start(b, bs):
            sbb, dbb, wbb, sem = bs
            base = pl.multiple_of(wid * EPT + b * BLK, 8)
            return (pltpu.async_copy(ei_hbm.at[pl.ds(base, BLK)], sbb, sem),
                    pltpu.async_copy(ei_hbm.at[pl.ds(E + base, BLK)], dbb, sem),
                    pltpu.async_copy(w_hbm.at[pl.ds(base, BLK)], wbb, sem))

        cps = start(0, bufs[0])

        pltpu.sync_copy(h_hbm, h_l.at[pl.ds(0, N)])

        zeros = jnp.zeros((16,), jnp.float32)

        @plsc.parallel_loop(0, NP, step=16, unroll=16)
        def _zero(i):
            acc[pl.ds(i, 16)] = zeros
        for b in range(NB):
            nxt = start(b + 1, bufs[(b + 1) % 2]) if b + 1 < NB else None
            for cp in cps:
                cp.wait()
            sbb, dbb, wbb, _ = bufs[b % 2]

            @plsc.parallel_loop(0, BLK, step=16, unroll=10)
            def _edges(e):
                si = sbb[pl.ds(e, 16)]
                di = dbb[pl.ds(e, 16)]
                wv = wbb[pl.ds(e, 16)]
                vals = plsc.load_gather(h_l, [si])
                plsc.addupdate_scatter(acc, [di], vals * wv)
            cps = nxt

        pltpu.sync_copy(acc, out_hbm.at[wid])

    return sc_kernel(h_pad, edge_index, ew)


def _ln_body(p_ref, g_ref, b_ref, o_ref):
    p = p_ref[...]
    h = jnp.sum(p, axis=0, keepdims=True)
    mu = jnp.sum(h) / float(N)
    var = jnp.sum(h * h) / float(N) - mu * mu
    hn = (h - mu) * lax.rsqrt(var + 1e-5)
    o_ref[...] = jnp.reshape(hn[:, :N] * g_ref[...] + b_ref[...], (N,))


def _layernorm(p, g2, b2):
    return pl.pallas_call(
        _ln_body,
        out_shape=jax.ShapeDtypeStruct((N,), jnp.float32),
    )(p, g2, b2)


def _head_body(hs_ref, d_ref, fw_ref, fb_ref, o_ref):
    v = hs_ref[...][:, 0:1]
    dv = d_ref[...][:, 0:1]
    sel = jnp.where(dv == 1.0, v, 0.0)
    mn = jnp.min(sel)
    mx = jnp.max(sel)
    sel = (sel - mn) / (mx - mn)
    m = jnp.sum(sel) / float(N // 10)
    y = m * fw_ref[0, 0] + fb_ref[0, 0]
    o_ref[...] = jnp.broadcast_to(jnp.maximum(y, 0.0), (1, 1))


def _head(hs, d2, fw, fb):
    return pl.pallas_call(
        _head_body,
        out_shape=jax.ShapeDtypeStruct((1, 1), jnp.float32),
    )(hs, d2, fw, fb)


def kernel(x, edge_index, edge_attr, W_retina, gumbel, ln_gamma, ln_beta,
           fc_w, fc_b, decision):
    x3 = x.reshape(T, 1, C)
    h_nz, ew = _retina(x3, W_retina, gumbel, edge_attr.reshape(T, 1, E // T))

    g2 = ln_gamma.reshape(1, N)
    b2 = ln_beta.reshape(1, N)

    ei_flat = edge_index.reshape(2 * E)
    p1 = _sc_pass(h_nz.reshape(N), ei_flat, ew)
    h1 = _layernorm(p1, g2, b2)
    p2 = _sc_pass(h1, ei_flat, ew)
    h2 = _layernorm(p2, g2, b2)

    hs = h2.reshape(N // 10, 10)
    d2 = decision.reshape(N // 10, 10)
    y = _head(hs, d2, fc_w, fc_b.reshape(1, 1))
    return y.reshape(1)


# speedup vs baseline: 1.2176x; 1.0254x over previous
import functools

import jax
import jax.numpy as jnp
from jax import lax
from jax.experimental import pallas as pl
from jax.experimental.pallas import tpu as pltpu
from jax.experimental.pallas import tpu_sc as plsc

N = 50000
T = 50
C = 1000
E = 3200000
NP = 50176
NTILES = 32
EPT = E // NTILES
BLK = 4000
NB = EPT // BLK
GRP = BLK // 16
EROWS = 3125


def _retina_body(x_ref, w_ref, g_ref, ea_ref, o_ref, ew_ref, h_s):
    t = pl.program_id(0)
    a = w_ref[0] + g_ref[0]
    m = jnp.max(a, axis=0, keepdims=True)
    p = jnp.exp(a - m)
    s = jnp.sum(p, axis=0, keepdims=True)
    v = x_ref[0] / s
    h_s[pl.ds(t, 1), :] = lax.dot_general(v, p, (((1,), (1,)), ((), ())),
                                          preferred_element_type=jnp.float32)
    ew_ref[0] = jnp.log1p(ea_ref[0])

    @pl.when(t == T - 1)
    def _():
        h = h_s[...]
        mask = h != 0.0
        cnt = jnp.sum(mask.astype(jnp.float32))
        mean = jnp.sum(jnp.where(mask, h, 0.0)) / cnt
        var = jnp.sum(jnp.where(mask, (h - mean) ** 2, 0.0)) / (cnt - 1.0)
        o_ref[...] = jnp.where(mask, (h - mean) / jnp.sqrt(var), h)


def _retina(x3, w, g, ea):
    return pl.pallas_call(
        _retina_body,
        grid=(T,),
        in_specs=[pl.BlockSpec((1, 1, C), lambda t: (t, 0, 0)),
                  pl.BlockSpec((1, C, C), lambda t: (t, 0, 0)),
                  pl.BlockSpec((1, C, C), lambda t: (t, 0, 0)),
                  pl.BlockSpec((1, 1, E // T), lambda t: (t, 0, 0))],
        out_specs=(pl.BlockSpec((T, C), lambda t: (0, 0)),
                   pl.BlockSpec((1, 1, E // T), lambda t: (t, 0, 0))),
        out_shape=(jax.ShapeDtypeStruct((T, C), jnp.float32),
                   jax.ShapeDtypeStruct((T, 1, E // T), jnp.float32)),
        scratch_shapes=[pltpu.VMEM((T, C), jnp.float32)],
    )(x3, w, g, ea)


def _sc_pass(h_pad, edge_index, ew):
    mesh = plsc.VectorSubcoreMesh(core_axis_name="c", subcore_axis_name="s",
                                  num_cores=2, num_subcores=16)

    @functools.partial(
        pl.kernel,
        out_type=jax.ShapeDtypeStruct((NTILES, NP), jnp.float32),
        mesh=mesh,
        compiler_params=pltpu.CompilerParams(needs_layout_passes=False),
        scratch_types=[
            pltpu.VMEM((NP,), jnp.float32),
            pltpu.VMEM((NP,), jnp.float32),
            pltpu.VMEM((BLK,), jnp.int32),
            pltpu.VMEM((BLK,), jnp.int32),
            pltpu.VMEM((BLK,), jnp.float32),
            pltpu.VMEM((BLK,), jnp.int32),
            pltpu.VMEM((BLK,), jnp.int32),
            pltpu.VMEM((BLK,), jnp.float32),
            pltpu.SemaphoreType.DMA,
            pltpu.SemaphoreType.DMA,
        ],
    )
    def sc_kernel(h_hbm, ei_hbm, w_hbm, out_hbm,
                  h_l, acc, sb0, db0, wb0, sb1, db1, wb1, sem0, sem1):
        cid = lax.axis_index("c")
        sid = lax.axis_index("s")
        wid = sid * 2 + cid

        bufs = ((sb0, db0, wb0, sem0), (sb1, db1, wb1, sem1))

        def start(b, bs):
            sbb, dbb, wbb, sem = bs
            base = pl.multiple_of(wid * EPT + b * BLK, 8)
            return (pltpu.async_copy(ei_hbm.at[pl.ds(base, BLK)], sbb, sem),
                    pltpu.async_copy(ei_hbm.at[pl.ds(E + base, BLK)], dbb, sem),
                    pltpu.async_copy(w_hbm.at[pl.ds(base, BLK)], wbb, sem))

        cps = start(0, bufs[0])

        pltpu.sync_copy(h_hbm, h_l.at[pl.ds(0, N)])

        zeros = jnp.zeros((16,), jnp.float32)

        @plsc.parallel_loop(0, NP, step=16, unroll=16)
        def _zero(i):
            acc[pl.ds(i, 16)] = zeros
        for b in range(NB):
            nxt = start(b + 1, bufs[(b + 1) % 2]) if b + 1 < NB else None
            for cp in cps:
                cp.wait()
            sbb, dbb, wbb, _ = bufs[b % 2]

            @plsc.parallel_loop(0, BLK, step=16, unroll=10)
            def _edges(e):
                si = sbb[pl.ds(e, 16)]
                di = dbb[pl.ds(e, 16)]
                wv = wbb[pl.ds(e, 16)]
                vals = plsc.load_gather(h_l, [si])
                plsc.addupdate_scatter(acc, [di], vals * wv)
            cps = nxt

        pltpu.sync_copy(acc, out_hbm.at[wid])

    return sc_kernel(h_pad, edge_index, ew)


def _ln_body(p_ref, g_ref, b_ref, o_ref):
    p = p_ref[...]
    h = jnp.sum(p, axis=0, keepdims=True)
    mu = jnp.sum(h) / float(N)
    var = jnp.sum(h * h) / float(N) - mu * mu
    hn = (h - mu) * lax.rsqrt(var + 1e-5)
    o_ref[...] = jnp.reshape(hn[:, :N] * g_ref[...] + b_ref[...], (N,))


def _layernorm(p, g2, b2):
    return pl.pallas_call(
        _ln_body,
        out_shape=jax.ShapeDtypeStruct((N,), jnp.float32),
    )(p, g2, b2)


def _head_body(h_ref, d_ref, fw_ref, fb_ref, o_ref):
    v = h_ref[...].reshape(1, N)
    dv = d_ref[...].reshape(1, N)
    idx = lax.broadcasted_iota(jnp.int32, (1, N), 1)
    on_dec = idx % 10 == 0
    sel = jnp.where(dv == 1.0, v, 0.0)
    mn = jnp.min(jnp.where(on_dec, sel, jnp.inf))
    mx = jnp.max(jnp.where(on_dec, sel, -jnp.inf))
    m = jnp.sum(jnp.where(on_dec, (sel - mn) / (mx - mn), 0.0)) / float(N // 10)
    y = m * fw_ref[0, 0] + fb_ref[0, 0]
    o_ref[...] = jnp.broadcast_to(jnp.maximum(y, 0.0), (1, 1))


def _head(h2, d, fw, fb):
    return pl.pallas_call(
        _head_body,
        out_shape=jax.ShapeDtypeStruct((1, 1), jnp.float32),
    )(h2, d, fw, fb)


def kernel(x, edge_index, edge_attr, W_retina, gumbel, ln_gamma, ln_beta,
           fc_w, fc_b, decision):
    x3 = x.reshape(T, 1, C)
    h_nz, ew2 = _retina(x3, W_retina, gumbel, edge_attr.reshape(T, 1, E // T))
    ew = ew2.reshape(E)

    g2 = ln_gamma.reshape(1, N)
    b2 = ln_beta.reshape(1, N)

    ei_flat = edge_index.reshape(2 * E)
    p1 = _sc_pass(h_nz.reshape(N), ei_flat, ew)
    h1 = _layernorm(p1, g2, b2)
    p2 = _sc_pass(h1, ei_flat, ew)
    h2 = _layernorm(p2, g2, b2)

    y = _head(h2, decision, fc_w, fc_b.reshape(1, 1))
    return y.reshape(1)
